# baseline (speedup 1.0000x reference)
import functools

import jax
import jax.numpy as jnp
from jax import lax
from jax.experimental import pallas as pl
from jax.experimental.pallas import tpu as pltpu
from jax.experimental.pallas import tpu_sc as plsc

N, SIZE1, SIZE2, OUT_SIZE, NNZ, C = 8192, 64, 64, 64, 128, 32
ROW = SIZE1 * C
NC, NS, L = 2, 16, 16
NW = NC * NS
SPW = N // NW
S = 8
NCHUNK = SPW // S
NG = NNZ // L


def _sc_body(x_hbm, y_hbm, scale_hbm, i1_hbm, i2_hbm, io_hbm, out_hbm,
             idxp_v, idx2_v, idxo_v, scale_v, scale_sp,
             xb0, xb1, yb0, yb1, ob0, ob1,
             semx0, semx1, semy0, semy1, semo0, semo1):
    wid = lax.axis_index("s") * NC + lax.axis_index("c")
    base = wid * SPW

    pltpu.sync_copy(i1_hbm, idxp_v)
    pltpu.sync_copy(i2_hbm, idx2_v)
    pltpu.sync_copy(io_hbm, idxo_v)
    pltpu.sync_copy(scale_hbm, scale_v)
    zeros16f = jnp.zeros((L,), jnp.float32)
    for v in range(NG):
        sl = pl.ds(L * v, L)
        idxp_v[sl] = idxp_v[sl] + idx2_v[sl] * 64 + idxo_v[sl] * 4096
        scvec = scale_v[sl]
        for t in range(L):
            scale_sp[pl.ds((L * v + t) * L, L)] = zeros16f + scvec[t]

    bufs = ((xb0, yb0, ob0, semx0, semy0, semo0),
            (xb1, yb1, ob1, semx1, semy1, semo1))

    def issue_loads(g, b):
        xb, yb, _, semx, semy, _ = bufs[b]
        row0 = base + g * S
        pltpu.async_copy(x_hbm.at[pl.ds(row0, S)], xb, semx)
        pltpu.async_copy(y_hbm.at[pl.ds(row0, S)], yb, semy)

    issue_loads(0, 0)
    issue_loads(1, 1)

    def chunk_pair(i, _):
        for b in range(2):
            g = 2 * i + b
            xb, yb, ob, semx, semy, semo = bufs[b]
            pltpu.make_async_copy(x_hbm.at[pl.ds(0, S)], xb, semx).wait()
            pltpu.make_async_copy(y_hbm.at[pl.ds(0, S)], yb, semy).wait()

            @pl.when(i >= 1)
            def _wait_out():
                pltpu.make_async_copy(
                    ob, out_hbm.at[pl.ds(0, S)], semo).wait()

            def zero_step(j, _):
                for s in range(S):
                    ob[s, pl.ds(j * L, L)] = zeros16f
                return 0

            lax.fori_loop(0, ROW // L, zero_step, 0)

            def group_step(v, _):
                pvec = idxp_v[pl.ds(v * L, L)]
                for t in range(L):
                    p = pvec[t]
                    o1 = (p & 63) * C
                    o2 = ((p >> 6) & 63) * C
                    oo = (p >> 12) * C
                    scv = scale_sp[pl.ds((v * L + t) * L, L)]
                    for s in range(S):
                        for h in range(C // L):
                            xv = xb[s, pl.ds(o1 + h * L, L)]
                            yv = yb[s, pl.ds(o2 + h * L, L)]
                            plsc.addupdate(
                                ob.at[s, pl.ds(oo + h * L, L)],
                                xv * yv * scv)
                return 0

            lax.fori_loop(0, NG, group_step, 0)

            row0 = base + g * S
            pltpu.async_copy(ob, out_hbm.at[pl.ds(row0, S)], semo)

            @pl.when(i <= NCHUNK // 2 - 2)
            def _next_loads():
                issue_loads(g + 2, b)

        return 0

    lax.fori_loop(0, NCHUNK // 2, chunk_pair, 0)

    for b in range(2):
        ob, semo = bufs[b][2], bufs[b][5]
        pltpu.make_async_copy(ob, out_hbm.at[pl.ds(0, S)], semo).wait()


@jax.jit
def kernel(x, y, scale, index1, index2, index_out):
    x2 = x.reshape(N, ROW)
    y2 = y.reshape(N, ROW)
    mesh = plsc.VectorSubcoreMesh(core_axis_name="c", subcore_axis_name="s")
    out2 = pl.kernel(
        _sc_body,
        out_type=jax.ShapeDtypeStruct((N, ROW), jnp.float32),
        mesh=mesh,
        scratch_types=[
            pltpu.VMEM((NNZ,), jnp.int32),
            pltpu.VMEM((NNZ,), jnp.int32),
            pltpu.VMEM((NNZ,), jnp.int32),
            pltpu.VMEM((NNZ,), jnp.float32),
            pltpu.VMEM((NNZ * L,), jnp.float32),
            pltpu.VMEM((S, ROW), jnp.float32),
            pltpu.VMEM((S, ROW), jnp.float32),
            pltpu.VMEM((S, ROW), jnp.float32),
            pltpu.VMEM((S, ROW), jnp.float32),
            pltpu.VMEM((S, ROW), jnp.float32),
            pltpu.VMEM((S, ROW), jnp.float32),
            pltpu.SemaphoreType.DMA,
            pltpu.SemaphoreType.DMA,
            pltpu.SemaphoreType.DMA,
            pltpu.SemaphoreType.DMA,
            pltpu.SemaphoreType.DMA,
            pltpu.SemaphoreType.DMA,
        ],
    )(x2, y2, scale, index1, index2, index_out)
    return out2.reshape(N, OUT_SIZE, C)


# speedup vs baseline: 2.1049x; 2.1049x over previous
import functools

import jax
import jax.numpy as jnp
from jax import lax
from jax.experimental import pallas as pl
from jax.experimental.pallas import tpu as pltpu
from jax.experimental.pallas import tpu_sc as plsc

N, SIZE1, SIZE2, OUT_SIZE, NNZ, C = 8192, 64, 64, 64, 128, 32
ROW = SIZE1 * C
NC, NS, L = 2, 16, 16
NW = NC * NS
SPW = N // NW
S = 8
NCHUNK = SPW // S
NG = NNZ // L


def _sc_body(x_hbm, y_hbm, scale_hbm, i1_hbm, i2_hbm, io_hbm, out_hbm,
             idxp_v, idx2_v, idxo_v, scale_v, scale_sp,
             xb0, xb1, yb0, yb1, ob0, ob1,
             semx0, semx1, semy0, semy1, semo0, semo1):
    wid = lax.axis_index("s") * NC + lax.axis_index("c")
    base = wid * SPW

    pltpu.sync_copy(i1_hbm, idxp_v)
    pltpu.sync_copy(i2_hbm, idx2_v)
    pltpu.sync_copy(io_hbm, idxo_v)
    pltpu.sync_copy(scale_hbm, scale_v)
    zeros16f = jnp.zeros((L,), jnp.float32)
    for v in range(NG):
        sl = pl.ds(L * v, L)
        idxp_v[sl] = idxp_v[sl] + idx2_v[sl] * 64 + idxo_v[sl] * 4096
        scvec = scale_v[sl]
        for t in range(L):
            scale_sp[pl.ds((L * v + t) * L, L)] = zeros16f + scvec[t]

    bufs = ((xb0, yb0, ob0, semx0, semy0, semo0),
            (xb1, yb1, ob1, semx1, semy1, semo1))

    def issue_loads(g, b):
        xb, yb, _, semx, semy, _ = bufs[b]
        row0 = base + g * S
        pltpu.async_copy(x_hbm.at[pl.ds(row0, S)], xb, semx)
        pltpu.async_copy(y_hbm.at[pl.ds(row0, S)], yb, semy)

    issue_loads(0, 0)
    issue_loads(1, 1)

    def chunk_pair(i, _):
        for b in range(2):
            g = 2 * i + b
            xb, yb, ob, semx, semy, semo = bufs[b]
            pltpu.make_async_copy(x_hbm.at[pl.ds(0, S)], xb, semx).wait()
            pltpu.make_async_copy(y_hbm.at[pl.ds(0, S)], yb, semy).wait()

            @pl.when(i >= 1)
            def _wait_out():
                pltpu.make_async_copy(
                    ob, out_hbm.at[pl.ds(0, S)], semo).wait()

            def zero_step(j, _):
                for s in range(S):
                    ob[s, pl.ds(j * L, L)] = zeros16f
                return 0

            lax.fori_loop(0, ROW // L, zero_step, 0)

            def group_step(v, _):
                pvec = idxp_v[pl.ds(v * L, L)]
                for t in range(L):
                    p = pvec[t]
                    o1 = (p & 63) * C
                    o2 = ((p >> 6) & 63) * C
                    oo = (p >> 12) * C
                    scv = scale_sp[pl.ds((v * L + t) * L, L)]
                    sh = [(s, h) for s in range(S) for h in range(C // L)]
                    xvs = [xb[s, pl.ds(o1 + h * L, L)] for s, h in sh]
                    yvs = [yb[s, pl.ds(o2 + h * L, L)] for s, h in sh]
                    prods = [xv * yv * scv for xv, yv in zip(xvs, yvs)]
                    for (s, h), pr in zip(sh, prods):
                        plsc.addupdate(ob.at[s, pl.ds(oo + h * L, L)], pr)
                return 0

            lax.fori_loop(0, NG, group_step, 0)

            row0 = base + g * S
            pltpu.async_copy(ob, out_hbm.at[pl.ds(row0, S)], semo)

            @pl.when(i <= NCHUNK // 2 - 2)
            def _next_loads():
                issue_loads(g + 2, b)

        return 0

    lax.fori_loop(0, NCHUNK // 2, chunk_pair, 0)

    for b in range(2):
        ob, semo = bufs[b][2], bufs[b][5]
        pltpu.make_async_copy(ob, out_hbm.at[pl.ds(0, S)], semo).wait()


@jax.jit
def kernel(x, y, scale, index1, index2, index_out):
    x2 = x.reshape(N, ROW)
    y2 = y.reshape(N, ROW)
    mesh = plsc.VectorSubcoreMesh(core_axis_name="c", subcore_axis_name="s")
    out2 = pl.kernel(
        _sc_body,
        out_type=jax.ShapeDtypeStruct((N, ROW), jnp.float32),
        mesh=mesh,
        compiler_params=pltpu.CompilerParams(
            use_tc_tiling_on_sc=False, needs_layout_passes=False),
        scratch_types=[
            pltpu.VMEM((NNZ,), jnp.int32),
            pltpu.VMEM((NNZ,), jnp.int32),
            pltpu.VMEM((NNZ,), jnp.int32),
            pltpu.VMEM((NNZ,), jnp.float32),
            pltpu.VMEM((NNZ * L,), jnp.float32),
            pltpu.VMEM((S, ROW), jnp.float32),
            pltpu.VMEM((S, ROW), jnp.float32),
            pltpu.VMEM((S, ROW), jnp.float32),
            pltpu.VMEM((S, ROW), jnp.float32),
            pltpu.VMEM((S, ROW), jnp.float32),
            pltpu.VMEM((S, ROW), jnp.float32),
            pltpu.SemaphoreType.DMA,
            pltpu.SemaphoreType.DMA,
            pltpu.SemaphoreType.DMA,
            pltpu.SemaphoreType.DMA,
            pltpu.SemaphoreType.DMA,
            pltpu.SemaphoreType.DMA,
        ],
    )(x2, y2, scale, index1, index2, index_out)
    return out2.reshape(N, OUT_SIZE, C)


# speedup vs baseline: 2.9509x; 1.4019x over previous
import functools

import jax
import jax.numpy as jnp
from jax import lax
from jax.experimental import pallas as pl
from jax.experimental.pallas import tpu as pltpu
from jax.experimental.pallas import tpu_sc as plsc

N, SIZE1, SIZE2, OUT_SIZE, NNZ, C = 8192, 64, 64, 64, 128, 32
ROW = SIZE1 * C
NC, NS, L = 2, 16, 16
NW = NC * NS
SPW = N // NW
S = 8
NCHUNK = SPW // S
NG = NNZ // L


def _sc_body(x_hbm, y_hbm, scale_hbm, i1_hbm, i2_hbm, io_hbm, out_hbm,
             idxp_v, idx2_v, idxo_v, scale_v, scale_sp,
             xb0, xb1, yb0, yb1, ob0, ob1,
             semx0, semx1, semy0, semy1, semo0, semo1):
    wid = lax.axis_index("s") * NC + lax.axis_index("c")
    base = wid * SPW

    pltpu.sync_copy(i1_hbm, idxp_v)
    pltpu.sync_copy(i2_hbm, idx2_v)
    pltpu.sync_copy(io_hbm, idxo_v)
    pltpu.sync_copy(scale_hbm, scale_v)
    zeros16f = jnp.zeros((L,), jnp.float32)
    for v in range(NG):
        sl = pl.ds(L * v, L)
        idxp_v[sl] = idxp_v[sl] + idx2_v[sl] * 64 + idxo_v[sl] * 4096
        scvec = scale_v[sl]
        for t in range(L):
            scale_sp[pl.ds((L * v + t) * L, L)] = zeros16f + scvec[t]

    bufs = ((xb0, yb0, ob0, semx0, semy0, semo0),
            (xb1, yb1, ob1, semx1, semy1, semo1))

    def issue_loads(g, b):
        xb, yb, _, semx, semy, _ = bufs[b]
        row0 = base + g * S
        pltpu.async_copy(x_hbm.at[pl.ds(row0, S)], xb, semx)
        pltpu.async_copy(y_hbm.at[pl.ds(row0, S)], yb, semy)

    issue_loads(0, 0)
    issue_loads(1, 1)

    def chunk_pair(i, _):
        for b in range(2):
            g = 2 * i + b
            xb, yb, ob, semx, semy, semo = bufs[b]
            pltpu.make_async_copy(x_hbm.at[pl.ds(0, S)], xb, semx).wait()
            pltpu.make_async_copy(y_hbm.at[pl.ds(0, S)], yb, semy).wait()

            @pl.when(i >= 1)
            def _wait_out():
                pltpu.make_async_copy(
                    ob, out_hbm.at[pl.ds(0, S)], semo).wait()

            def zero_step(j, _):
                for s in range(S):
                    ob[s, pl.ds(j * L, L)] = zeros16f
                return 0

            lax.fori_loop(0, ROW // L, zero_step, 0)

            def group_step(v, _):
                pvec = idxp_v[pl.ds(v * L, L)]
                for t in range(L):
                    p = pvec[t]
                    o1 = (p & 63) * C
                    o2 = ((p >> 6) & 63) * C
                    oo = (p >> 12) * C
                    scv = scale_sp[pl.ds((v * L + t) * L, L)]
                    sh = [(s, h) for s in range(S) for h in range(C // L)]
                    xvs = [xb[s, pl.ds(o1 + h * L, L)] for s, h in sh]
                    yvs = [yb[s, pl.ds(o2 + h * L, L)] for s, h in sh]
                    prods = [xv * yv * scv for xv, yv in zip(xvs, yvs)]
                    for (s, h), pr in zip(sh, prods):
                        plsc.addupdate(ob.at[s, pl.ds(oo + h * L, L)], pr)
                return 0

            lax.fori_loop(0, NG, group_step, 0)

            row0 = base + g * S
            pltpu.async_copy(ob, out_hbm.at[pl.ds(row0, S)], semo)

            @pl.when(i <= NCHUNK // 2 - 2)
            def _next_loads():
                issue_loads(g + 2, b)

        return 0

    lax.fori_loop(0, NCHUNK // 2, chunk_pair, 0)

    for b in range(2):
        ob, semo = bufs[b][2], bufs[b][5]
        pltpu.make_async_copy(ob, out_hbm.at[pl.ds(0, S)], semo).wait()


@jax.jit
def kernel(x, y, scale, index1, index2, index_out):
    x2 = x.reshape(N, ROW)
    y2 = y.reshape(N, ROW)
    mesh = plsc.VectorSubcoreMesh(core_axis_name="c", subcore_axis_name="s")
    out2 = pl.kernel(
        _sc_body,
        out_type=jax.ShapeDtypeStruct((N, ROW), jnp.float32),
        mesh=mesh,
        scratch_types=[
            pltpu.VMEM((NNZ,), jnp.int32),
            pltpu.VMEM((NNZ,), jnp.int32),
            pltpu.VMEM((NNZ,), jnp.int32),
            pltpu.VMEM((NNZ,), jnp.float32),
            pltpu.VMEM((NNZ * L,), jnp.float32),
            pltpu.VMEM((S, ROW), jnp.float32),
            pltpu.VMEM((S, ROW), jnp.float32),
            pltpu.VMEM((S, ROW), jnp.float32),
            pltpu.VMEM((S, ROW), jnp.float32),
            pltpu.VMEM((S, ROW), jnp.float32),
            pltpu.VMEM((S, ROW), jnp.float32),
            pltpu.SemaphoreType.DMA,
            pltpu.SemaphoreType.DMA,
            pltpu.SemaphoreType.DMA,
            pltpu.SemaphoreType.DMA,
            pltpu.SemaphoreType.DMA,
            pltpu.SemaphoreType.DMA,
        ],
    )(x2, y2, scale, index1, index2, index_out)
    return out2.reshape(N, OUT_SIZE, C)
